# baseline (speedup 1.0000x reference)
import functools

import jax
import jax.numpy as jnp
from jax import lax
from jax.experimental import pallas as pl
from jax.experimental.pallas import tpu as pltpu
from jax.experimental.pallas import tpu_sc as plsc

BLOCK = 8192
EMBED = 1024

_info = plsc.get_sparse_core_info()
_NC, _NS = _info.num_cores, _info.num_subcores
_NW = _NC * _NS
_ROWS_PER_W = BLOCK // _NW
_CHUNK = 32
_NSTEPS = _ROWS_PER_W // _CHUNK


def _copy_body(pe_hbm, out_hbm, sem):
    wid = lax.axis_index("s") * _NC + lax.axis_index("c")
    base = wid * _ROWS_PER_W
    pltpu.async_copy(
        pe_hbm.at[pl.ds(base, _ROWS_PER_W)],
        out_hbm.at[pl.ds(base, _ROWS_PER_W)], sem).wait()


def _sc_copy(pe):
    mesh = plsc.VectorSubcoreMesh(core_axis_name="c", subcore_axis_name="s")
    return pl.kernel(
        _copy_body,
        mesh=mesh,
        out_type=jax.ShapeDtypeStruct((BLOCK, EMBED), jnp.float32),
        scratch_types=[
            pltpu.SemaphoreType.DMA,
        ],
    )(pe)


def kernel(x, pe):
    return _sc_copy(pe)


# speedup vs baseline: 23.1848x; 23.1848x over previous
import functools

import jax
import jax.numpy as jnp
from jax import lax
from jax.experimental import pallas as pl
from jax.experimental.pallas import tpu as pltpu
from jax.experimental.pallas import tpu_sc as plsc

BLOCK = 8192
EMBED = 1024

_info = plsc.get_sparse_core_info()
_NC, _NS = _info.num_cores, _info.num_subcores
_NW = _NC * _NS
_ROWS_PER_W = BLOCK // _NW
_CHUNK = 32
_NSTEPS = _ROWS_PER_W // _CHUNK


def _copy_body(pe_hbm, out_hbm, buf0, buf1, sem_in, sem_out):
    wid = lax.axis_index("s") * _NC + lax.axis_index("c")
    base = wid * _ROWS_PER_W
    bufs = (buf0, buf1)

    def start_in(i):
        return pltpu.async_copy(
            pe_hbm.at[pl.ds(base + i * _CHUNK, _CHUNK)], bufs[i % 2], sem_in)

    def start_out(i):
        return pltpu.async_copy(
            bufs[i % 2], out_hbm.at[pl.ds(base + i * _CHUNK, _CHUNK)], sem_out)

    copies_in = [None] * _NSTEPS
    copies_out = [None] * _NSTEPS
    copies_in[0] = start_in(0)
    for i in range(_NSTEPS):
        if i > 0:
            copies_out[i - 1].wait()
        copies_in[i].wait()
        copies_out[i] = start_out(i)
        if i + 1 < _NSTEPS:
            copies_in[i + 1] = start_in(i + 1)
    copies_out[_NSTEPS - 1].wait()


def _sc_copy(pe):
    mesh = plsc.VectorSubcoreMesh(core_axis_name="c", subcore_axis_name="s")
    return pl.kernel(
        _copy_body,
        mesh=mesh,
        out_type=jax.ShapeDtypeStruct((BLOCK, EMBED), jnp.float32),
        scratch_types=[
            pltpu.VMEM((_CHUNK, EMBED), jnp.float32),
            pltpu.VMEM((_CHUNK, EMBED), jnp.float32),
            pltpu.SemaphoreType.DMA,
            pltpu.SemaphoreType.DMA,
        ],
    )(pe)


def kernel(x, pe):
    return _sc_copy(pe)


# speedup vs baseline: 24.3806x; 1.0516x over previous
import functools

import jax
import jax.numpy as jnp
from jax import lax
from jax.experimental import pallas as pl
from jax.experimental.pallas import tpu as pltpu
from jax.experimental.pallas import tpu_sc as plsc

BLOCK = 8192
EMBED = 1024

_info = plsc.get_sparse_core_info()
_NC, _NS = _info.num_cores, _info.num_subcores
_NW = _NC * _NS
_ROWS_PER_W = BLOCK // _NW
_CHUNK = 16
_NSTEPS = _ROWS_PER_W // _CHUNK
_NBUF = 6
_AHEAD = 4


def _copy_body(pe_hbm, out_hbm, *rest):
    bufs = rest[:_NBUF]
    sem_in, sem_out = rest[_NBUF], rest[_NBUF + 1]
    wid = lax.axis_index("s") * _NC + lax.axis_index("c")
    base = wid * _ROWS_PER_W

    def start_in(i):
        return pltpu.async_copy(
            pe_hbm.at[pl.ds(base + i * _CHUNK, _CHUNK)], bufs[i % _NBUF], sem_in)

    def start_out(i):
        return pltpu.async_copy(
            bufs[i % _NBUF], out_hbm.at[pl.ds(base + i * _CHUNK, _CHUNK)], sem_out)

    copies_in = [None] * _NSTEPS
    copies_out = [None] * _NSTEPS
    out_waited = [False] * _NSTEPS
    for i in range(_AHEAD):
        copies_in[i] = start_in(i)
    for i in range(_NSTEPS):
        copies_in[i].wait()
        copies_out[i] = start_out(i)
        j = i + _AHEAD
        if j < _NSTEPS:
            prev = j - _NBUF
            if prev >= 0:
                copies_out[prev].wait()
                out_waited[prev] = True
            copies_in[j] = start_in(j)
    for i in range(_NSTEPS):
        if not out_waited[i]:
            copies_out[i].wait()


def _sc_copy(pe):
    mesh = plsc.VectorSubcoreMesh(core_axis_name="c", subcore_axis_name="s")
    return pl.kernel(
        _copy_body,
        mesh=mesh,
        out_type=jax.ShapeDtypeStruct((BLOCK, EMBED), jnp.float32),
        scratch_types=(
            [pltpu.VMEM((_CHUNK, EMBED), jnp.float32) for _ in range(_NBUF)]
            + [pltpu.SemaphoreType.DMA, pltpu.SemaphoreType.DMA]
        ),
    )(pe)


def kernel(x, pe):
    return _sc_copy(pe)
